# Initial kernel scaffold; baseline (speedup 1.0000x reference)
import jax
import jax.numpy as jnp
from jax.experimental import pallas as pl


def kernel(x, edge_index, W1l, b1l, W1r, W2l, b2l, W2r, W3l, b3l, W3r):
    raise NotImplementedError("write your pallas kernel here")


# speedup vs baseline: 3.3241x; 3.3241x over previous
import jax
import jax.numpy as jnp
from jax import lax
from jax.experimental import pallas as pl
from jax.experimental.pallas import tpu as pltpu
from jax.experimental.pallas import tpu_sc as plsc

N = 10000
NP = 10240
E = 320000
D = 128
CH = 64
NC, NS = 2, 16
NW = NC * NS
CPW = -(-(E // CH) // NW)
EP = CPW * NW * CH
RPT = NP // NS


def _make_sc_agg(gather: bool):
  mesh = plsc.VectorSubcoreMesh(core_axis_name="c", subcore_axis_name="s")

  def body(*refs):
    if gather:
      (table, src, dst, zseed, acc_out,
       sidx, didx, rows, acc, sem) = refs
      oseed = None
    else:
      (dst, zseed, oseed, acc_out,
       didx, rows, acc, sem) = refs
      sidx = src = table = None
    c = lax.axis_index("c")
    s = lax.axis_index("s")
    wid = s * NC + c

    for k in range(CH // 8):
      pltpu.sync_copy(zseed, rows.at[pl.ds(k * 8, 8)])
    for k in range(RPT // CH):
      pltpu.sync_copy(rows, acc.at[pl.ds(s * RPT + k * CH, CH)])
    if not gather:
      for k in range(CH // 8):
        pltpu.sync_copy(oseed, rows.at[pl.ds(k * 8, 8)])
    plsc.subcore_barrier()

    start = wid * CPW

    if gather:
      def step(i, carry):
        base = pl.multiple_of(i * CH, CH)
        pltpu.sync_copy(src.at[pl.ds(base, CH)], sidx)
        pltpu.sync_copy(dst.at[pl.ds(base, CH)], didx)
        pltpu.async_copy(table.at[sidx], rows, sem).wait()
        pltpu.sync_copy(rows, acc.at[didx], add=True)
        return carry
    else:
      def step(i, carry):
        base = pl.multiple_of(i * CH, CH)
        pltpu.sync_copy(dst.at[pl.ds(base, CH)], didx)
        pltpu.sync_copy(rows, acc.at[didx], add=True)
        return carry

    lax.fori_loop(start, start + CPW, step, 0)
    plsc.subcore_barrier()

    for k in range(RPT // CH):
      off = s * RPT + k * CH
      pltpu.sync_copy(acc.at[pl.ds(off, CH)], rows)
      pltpu.sync_copy(rows, acc_out.at[c, pl.ds(off, CH)])

  scratch = [
      pltpu.VMEM((CH,), jnp.int32),
      pltpu.VMEM((CH, D), jnp.float32),
      pltpu.VMEM_SHARED((NP, D), jnp.float32),
      pltpu.SemaphoreType.DMA,
  ]
  if gather:
    scratch.insert(0, pltpu.VMEM((CH,), jnp.int32))

  return pl.kernel(body,
                   out_type=jax.ShapeDtypeStruct((NC, NP, D), jnp.float32),
                   mesh=mesh, scratch_types=scratch)


_sc_agg = _make_sc_agg(True)
_sc_cnt = _make_sc_agg(False)


def _make_tc_layer(relu: bool):
  R = 1280

  def body(a0, a1, c0, c1, h, wl, wr, b, o):
    cnt = c0[:, :1] + c1[:, :1]
    inv = 1.0 / jnp.maximum(cnt, 1.0)
    agg = (a0[...] + a1[...]) * inv
    y = jnp.dot(agg, wl[...], preferred_element_type=jnp.float32)
    y = y + jnp.dot(h[...], wr[...], preferred_element_type=jnp.float32)
    y = y + b[...]
    if relu:
      y = jnp.maximum(y, 0.0)
    o[...] = y

  row = lambda i: (i, 0)
  zero = lambda i: (0, 0)
  return pl.pallas_call(
      body,
      grid=(NP // R,),
      in_specs=[
          pl.BlockSpec((R, D), row),
          pl.BlockSpec((R, D), row),
          pl.BlockSpec((R, D), row),
          pl.BlockSpec((R, D), row),
          pl.BlockSpec((R, D), row),
          pl.BlockSpec((D, D), zero),
          pl.BlockSpec((D, D), zero),
          pl.BlockSpec((1, D), zero),
      ],
      out_specs=pl.BlockSpec((R, D), row),
      out_shape=jax.ShapeDtypeStruct((NP, D), jnp.float32),
  )


_tc_relu = _make_tc_layer(True)
_tc_lin = _make_tc_layer(False)


@jax.jit
def kernel(x, edge_index, W1l, b1l, W1r, W2l, b2l, W2r, W3l, b3l, W3r):
  src = jnp.pad(edge_index[0], (0, EP - E))
  dst = jnp.pad(edge_index[1], (0, EP - E), constant_values=NP - 1)
  xp = jnp.pad(x, ((0, NP - N), (0, 0)))
  zseed = jnp.zeros((8, D), jnp.float32)
  oseed = jnp.ones((8, D), jnp.float32)

  cntp = _sc_cnt(dst, zseed, oseed)
  acc1 = _sc_agg(xp, src, dst, zseed)
  h1 = _tc_relu(acc1[0], acc1[1], cntp[0], cntp[1], xp,
                W1l, W1r, b1l.reshape(1, D))
  acc2 = _sc_agg(h1, src, dst, zseed)
  h2 = _tc_relu(acc2[0], acc2[1], cntp[0], cntp[1], h1,
                W2l, W2r, b2l.reshape(1, D))
  acc3 = _sc_agg(h2, src, dst, zseed)
  out = _tc_lin(acc3[0], acc3[1], cntp[0], cntp[1], h2,
                W3l, W3r, b3l.reshape(1, D))
  return out[:N]
